# baseline (speedup 1.0000x reference)
import functools

import jax
import jax.numpy as jnp
from jax import lax
from jax.experimental import pallas as pl
from jax.experimental.pallas import tpu as pltpu
from jax.experimental.pallas import tpu_sc as plsc

B = 4096
L = 200
D = 64
N = B * L
NW = 32
PER_W = N // NW
TPC = 20
CHUNK = 16 * TPC
CPT = (L // 2) // TPC
NCHUNK = PER_W // CHUNK
NBUF = 4


def _gather_body(idx_hbm, table_hbm, out_hbm, idx_raw, pidx, rows_v,
                 gsem, wsem):
    wid = lax.axis_index("s") * 2 + lax.axis_index("c")
    base = wid * PER_W

    pltpu.sync_copy(idx_hbm.at[wid], idx_raw)

    lane = lax.iota(jnp.int32, 16)
    v0 = (lane >> 1) * L + (lane & 1)

    def build(c, b):
        ii = c // CPT
        j0 = (c % CPT) * TPC
        off = (8 * L) * ii + 2 * j0
        dst = pidx.at[b]
        for q in range(TPC):
            vals = plsc.load_gather(idx_raw, [v0 + (off + 2 * q)])
            dst[pl.ds(q * 16, 16)] = vals

    def gather(b):
        return pltpu.make_async_copy(
            table_hbm.at[pidx.at[b]], rows_v.at[b], gsem.at[b])

    def write(i, b):
        return pltpu.make_async_copy(
            rows_v.at[b], out_hbm.at[pl.ds(base + i * CHUNK, CHUNK)],
            wsem.at[b])

    for b in range(NBUF):
        build(b, b)
        gather(b).start()

    def group(g, carry):
        for b in range(NBUF):
            i = g + b
            gather(b).wait()
            write(i, b).start()
        for b in range(NBUF):
            i = g + b
            nxt = i + NBUF

            @pl.when(nxt < NCHUNK)
            def _():
                write(i, b).wait()
                build(nxt, b)
                gather(b).start()

        return carry

    lax.fori_loop(0, NCHUNK // NBUF, lambda k, c: group(k * NBUF, c), 0)

    for b in range(NBUF):
        write(NCHUNK - NBUF + b, b).wait()


V = 1000000
NT = V // 8
TPW = NT // NW
XTRA = NT - TPW * NW
CT = 31
NCH = TPW // CT
CR = CT * 8


def _depad_body(tab_hbm, out_hbm, pin0, pin1, pout0, pout1,
                g0, g1, w0, w1):
    wid = lax.axis_index("s") * 2 + lax.axis_index("c")
    base = TPW * wid + jnp.minimum(wid, XTRA)

    def rd(t0, pin_b, sem):
        return pltpu.make_async_copy(
            tab_hbm.at[pl.ds(8 * t0, CR), :], pin_b, sem)

    def wr(t0, pout_b, sem):
        return pltpu.make_async_copy(
            pout_b, out_hbm.at[pl.ds(512 * t0, CR * D)], sem)

    def compact(pin_b, pout_b, nrows):
        for r in range(nrows):
            for q in range(4):
                pout_b[pl.ds(r * 64 + 16 * q, 16)] = (
                    pin_b[r, pl.ds(16 * q, 16)])

    bufs = [(pin0, pout0, g0, w0), (pin1, pout1, g1, w1)]
    rd(base, pin0, g0).start()
    rd(base + CT, pin1, g1).start()

    def group(g, carry):
        for b in range(2):
            k = 2 * g + b
            t0 = base + k * CT
            pin_b, pout_b, gb, wb = bufs[b]
            rd(t0, pin_b, gb).wait()

            @pl.when(k >= 2)
            def _():
                wr(t0 - 2 * CT, pout_b, wb).wait()

            compact(pin_b, pout_b, CR)
            wr(t0, pout_b, wb).start()

            @pl.when(k + 2 < NCH)
            def _():
                rd(t0 + 2 * CT, pin_b, gb).start()

        return carry

    lax.fori_loop(0, NCH // 2, group, 0)
    wr(base + (NCH - 2) * CT, pout0, w0).wait()
    wr(base + (NCH - 1) * CT, pout1, w1).wait()

    @pl.when(wid < XTRA)
    def _():
        t = base + TPW
        pltpu.sync_copy(tab_hbm.at[pl.ds(8 * t, 8), :],
                        pin0.at[pl.ds(0, 8), :])
        compact(pin0, pout0, 8)
        pltpu.sync_copy(pout0.at[pl.ds(0, 8 * D)],
                        out_hbm.at[pl.ds(512 * t, 8 * D)])


_depad = functools.partial(
    pl.kernel,
    out_type=jax.ShapeDtypeStruct((V * D,), jnp.float32),
    mesh=plsc.VectorSubcoreMesh(core_axis_name="c", subcore_axis_name="s"),
    scratch_types=[
        pltpu.VMEM((CR, D), jnp.float32),
        pltpu.VMEM((CR, D), jnp.float32),
        pltpu.VMEM((CR * D,), jnp.float32),
        pltpu.VMEM((CR * D,), jnp.float32),
        pltpu.SemaphoreType.DMA,
        pltpu.SemaphoreType.DMA,
        pltpu.SemaphoreType.DMA,
        pltpu.SemaphoreType.DMA,
    ],
    compiler_params=pltpu.CompilerParams(
        use_tc_tiling_on_sc=True, needs_layout_passes=False),
)(_depad_body)


_gather = functools.partial(
    pl.kernel,
    out_type=jax.ShapeDtypeStruct((N, D), jnp.float32),
    mesh=plsc.VectorSubcoreMesh(core_axis_name="c", subcore_axis_name="s"),
    scratch_types=[
        pltpu.VMEM((PER_W,), jnp.int32),
        pltpu.VMEM((NBUF, CHUNK), jnp.int32),
        pltpu.VMEM((NBUF, CHUNK, D), jnp.float32),
        pltpu.SemaphoreType.DMA((NBUF,)),
        pltpu.SemaphoreType.DMA((NBUF,)),
    ],
    compiler_params=pltpu.CompilerParams(
        use_tc_tiling_on_sc=False, needs_layout_passes=False),
)(_gather_body)


@jax.jit
def kernel(inputs, table):
    idx = inputs.reshape(NW, PER_W)
    tab = _depad(table).reshape(V, D)
    rows = _gather(idx, tab)
    out4 = rows.reshape(B // 8, L // 2, 8, 2 * D)
    return out4.transpose(0, 2, 1, 3).reshape(B, L * D)


# speedup vs baseline: 1.1759x; 1.1759x over previous
import functools

import jax
import jax.numpy as jnp
from jax import lax
from jax.experimental import pallas as pl
from jax.experimental.pallas import tpu as pltpu
from jax.experimental.pallas import tpu_sc as plsc

B = 4096
L = 200
D = 64
N = B * L
NW = 32
PER_W = N // NW
TPC = 20
CHUNK = 16 * TPC
CPT = (L // 2) // TPC
NCHUNK = PER_W // CHUNK
NBUF = 4


def _gather_body(idx_hbm, table_hbm, out_hbm, idx_raw, pidx, rows_v,
                 gsem, wsem):
    wid = lax.axis_index("s") * 2 + lax.axis_index("c")
    base = wid * PER_W

    pltpu.sync_copy(idx_hbm.at[wid], idx_raw)

    lane = lax.iota(jnp.int32, 16)
    v0 = (lane >> 1) * L + (lane & 1)

    def build(c, b):
        ii = c // CPT
        j0 = (c % CPT) * TPC
        off = (8 * L) * ii + 2 * j0
        dst = pidx.at[b]
        for q in range(TPC):
            vals = plsc.load_gather(idx_raw, [v0 + (off + 2 * q)])
            dst[pl.ds(q * 16, 16)] = vals

    def gather(b):
        return pltpu.make_async_copy(
            table_hbm.at[pidx.at[b]], rows_v.at[b], gsem.at[b])

    def write(i, b):
        return pltpu.make_async_copy(
            rows_v.at[b], out_hbm.at[pl.ds(base + i * CHUNK, CHUNK)],
            wsem.at[b])

    for b in range(NBUF):
        build(b, b)
        gather(b).start()

    def group(g, carry):
        for b in range(NBUF):
            i = g + b
            gather(b).wait()
            write(i, b).start()
        for b in range(NBUF):
            i = g + b
            nxt = i + NBUF

            @pl.when(nxt < NCHUNK)
            def _():
                write(i, b).wait()
                build(nxt, b)
                gather(b).start()

        return carry

    lax.fori_loop(0, NCHUNK // NBUF, lambda k, c: group(k * NBUF, c), 0)

    for b in range(NBUF):
        write(NCHUNK - NBUF + b, b).wait()


V = 1000000
NT = V // 8
TPW = NT // NW
XTRA = NT - TPW * NW
CT = 31
NCH = TPW // CT
CR = CT * 8


def _depad_body(tab_hbm, out_hbm, pin0, pin1, pout0, pout1,
                g0, g1, w0, w1):
    wid = lax.axis_index("s") * 2 + lax.axis_index("c")
    base = TPW * wid + jnp.minimum(wid, XTRA)

    def rd(t0, pin_b, sem):
        return pltpu.make_async_copy(
            tab_hbm.at[pl.ds(t0, CT), :, :], pin_b, sem)

    def wr(t0, pout_b, sem):
        return pltpu.make_async_copy(
            pout_b, out_hbm.at[pl.ds(512 * t0, CR * D)], sem)

    def compact(pin_b, pout_b, nrows):
        for r in range(nrows):
            for q in range(4):
                pout_b[pl.ds(r * 64 + 16 * q, 16)] = (
                    pin_b[r // 8, r % 8, pl.ds(16 * q, 16)])

    bufs = [(pin0, pout0, g0, w0), (pin1, pout1, g1, w1)]
    rd(base, pin0, g0).start()
    rd(base + CT, pin1, g1).start()

    def group(g, carry):
        for b in range(2):
            k = 2 * g + b
            t0 = base + k * CT
            pin_b, pout_b, gb, wb = bufs[b]
            rd(t0, pin_b, gb).wait()

            @pl.when(k >= 2)
            def _():
                wr(t0 - 2 * CT, pout_b, wb).wait()

            compact(pin_b, pout_b, CR)
            wr(t0, pout_b, wb).start()

            @pl.when(k + 2 < NCH)
            def _():
                rd(t0 + 2 * CT, pin_b, gb).start()

        return carry

    lax.fori_loop(0, NCH // 2, group, 0)
    wr(base + (NCH - 2) * CT, pout0, w0).wait()
    wr(base + (NCH - 1) * CT, pout1, w1).wait()

    @pl.when(wid < XTRA)
    def _():
        t = base + TPW
        pltpu.sync_copy(tab_hbm.at[pl.ds(t, 1), :, :],
                        pin0.at[pl.ds(0, 1), :, :])
        compact(pin0, pout0, 8)
        pltpu.sync_copy(pout0.at[pl.ds(0, 8 * D)],
                        out_hbm.at[pl.ds(512 * t, 8 * D)])


_depad = functools.partial(
    pl.kernel,
    out_type=jax.ShapeDtypeStruct((V * D,), jnp.float32),
    mesh=plsc.VectorSubcoreMesh(core_axis_name="c", subcore_axis_name="s"),
    scratch_types=[
        pltpu.VMEM((CT, 8, D), jnp.float32),
        pltpu.VMEM((CT, 8, D), jnp.float32),
        pltpu.VMEM((CR * D,), jnp.float32),
        pltpu.VMEM((CR * D,), jnp.float32),
        pltpu.SemaphoreType.DMA,
        pltpu.SemaphoreType.DMA,
        pltpu.SemaphoreType.DMA,
        pltpu.SemaphoreType.DMA,
    ],
    compiler_params=pltpu.CompilerParams(
        use_tc_tiling_on_sc=True, needs_layout_passes=False),
)(_depad_body)


_gather = functools.partial(
    pl.kernel,
    out_type=jax.ShapeDtypeStruct((N, D), jnp.float32),
    mesh=plsc.VectorSubcoreMesh(core_axis_name="c", subcore_axis_name="s"),
    scratch_types=[
        pltpu.VMEM((PER_W,), jnp.int32),
        pltpu.VMEM((NBUF, CHUNK), jnp.int32),
        pltpu.VMEM((NBUF, CHUNK, D), jnp.float32),
        pltpu.SemaphoreType.DMA((NBUF,)),
        pltpu.SemaphoreType.DMA((NBUF,)),
    ],
    compiler_params=pltpu.CompilerParams(
        use_tc_tiling_on_sc=False, needs_layout_passes=False),
)(_gather_body)


@jax.jit
def kernel(inputs, table):
    idx = inputs.reshape(NW, PER_W)
    tab = _depad(table.reshape(NT, 8, D)).reshape(V, D)
    rows = _gather(idx, tab)
    out4 = rows.reshape(B // 8, L // 2, 8, 2 * D)
    return out4.transpose(0, 2, 1, 3).reshape(B, L * D)
